# baseline (speedup 1.0000x reference)
import functools

import jax
import jax.numpy as jnp
from jax import lax
from jax.experimental import pallas as pl
from jax.experimental.pallas import tpu as pltpu
from jax.experimental.pallas import tpu_sc as plsc

B = 16
N = 2048
K = 16
N_SPACE = 4
N_ALL = 16
NUM_CLASSES = 10
LAYER_DIMS = [15, 20, 25, 30, 35, 40]

_NC, _NS = 2, 16
_NW = _NC * _NS
_NODES = B * N
_CHN = 64
_NCHUNK = _NODES // _NW // _CHN


def _make_sc_gather(c_pad, total_nodes, nchunk, chn):
    mesh = plsc.VectorSubcoreMesh(core_axis_name="c", subcore_axis_name="s")
    npw = total_nodes // _NW

    @functools.partial(
        pl.kernel,
        out_type=jax.ShapeDtypeStruct((total_nodes, K * c_pad), jnp.float32),
        mesh=mesh,
        scratch_types=[
            pltpu.VMEM((nchunk, K, chn), jnp.int32),
            pltpu.VMEM((2, K, chn, c_pad), jnp.float32),
            pltpu.SemaphoreType.DMA,
            pltpu.SemaphoreType.DMA,
        ],
        compiler_params=pltpu.CompilerParams(use_tc_tiling_on_sc=False),
    )
    def gather_k(table_hbm, idx_hbm, out_hbm, idx_v, rows_v, sem_g, sem_w):
        wid = lax.axis_index("s") * _NC + lax.axis_index("c")
        base = wid * npw
        pltpu.sync_copy(idx_hbm.at[wid], idx_v)

        def wb_chunk(j, p, issue):
            for k in range(K):
                c = pltpu.make_async_copy(
                    rows_v.at[p, k],
                    out_hbm.at[pl.ds(base + j * chn, chn),
                               pl.ds(k * c_pad, c_pad)],
                    sem_w)
                if issue:
                    c.start()
                else:
                    c.wait()

        def body(j, carry):
            p = lax.rem(j, 2)

            @pl.when(j >= 2)
            def _():
                wb_chunk(j - 2, p, issue=False)

            gathers = []
            for k in range(K):
                gathers.append(pltpu.async_copy(
                    table_hbm.at[idx_v.at[j, k]], rows_v.at[p, k], sem_g))
            for g in gathers:
                g.wait()
            wb_chunk(j, p, issue=True)
            return carry

        lax.fori_loop(0, nchunk, body, 0)
        for j in (nchunk - 2, nchunk - 1):
            wb_chunk(j, j % 2, issue=False)

    return gather_k


def _mm_kernel(pre_ref, w_ref, b_ref, out_ref):
    pre = pre_ref[0]
    acc = jnp.dot(pre, w_ref[...], preferred_element_type=jnp.float32)
    out_ref[0] = jnp.maximum(acc + b_ref[...], 0.0)


def _layer_matmul(pre, W, b):
    Bb, n, kc = pre.shape
    d = W.shape[1]
    return pl.pallas_call(
        _mm_kernel,
        grid=(Bb,),
        in_specs=[
            pl.BlockSpec((1, n, kc), lambda i: (i, 0, 0)),
            pl.BlockSpec((kc, d), lambda i: (0, 0)),
            pl.BlockSpec((1, d), lambda i: (0, 0)),
        ],
        out_specs=pl.BlockSpec((1, n, d), lambda i: (i, 0, 0)),
        out_shape=jax.ShapeDtypeStruct((Bb, n, d), jnp.float32),
    )(pre, W, b)


def _head_kernel(feat_ref, ne_ref, w1_ref, b1_ref, w2_ref, b2_ref,
                 w3_ref, b3_ref, out_ref):
    feat = feat_ref[...]
    bb, n, _ = feat_ref.shape
    sq = ne_ref[...]
    pos = jax.lax.broadcasted_iota(jnp.int32, (bb, n), 1)
    mask = (pos < sq).astype(jnp.float32)
    s = jnp.sum(feat * mask[..., None], axis=1)
    flattened = jnp.clip(s / sq.astype(jnp.float32), -1e9, 1e9)
    h1 = jnp.maximum(jnp.dot(flattened, w1_ref[...],
                             preferred_element_type=jnp.float32)
                     + b1_ref[...], 0.0)
    h2 = jnp.maximum(jnp.dot(h1, w2_ref[...],
                             preferred_element_type=jnp.float32)
                     + b2_ref[...], 0.0)
    out_ref[...] = jnp.dot(h2, w3_ref[...],
                           preferred_element_type=jnp.float32) + b3_ref[...]


def _head(feat, num_entries, w1, b1, w2, b2, w3, b3):
    Bb = feat.shape[0]
    return pl.pallas_call(
        _head_kernel,
        out_shape=jax.ShapeDtypeStruct((Bb, NUM_CLASSES), jnp.float32),
    )(feat, num_entries, w1, b1, w2, b2, w3, b3)


def _pad8(c):
    return -(-c // 8) * 8


def _pad_weight(W, c_in, c_pad, w_out, w_pad):
    W3 = W.reshape(K, c_in, w_out)
    W3 = jnp.pad(W3, ((0, 0), (0, c_pad - c_in), (0, w_pad - w_out)))
    return W3.reshape(K * c_pad, w_pad)


def kernel(space_features, all_features, neighbors_matrix, num_entries, params):
    nbr = neighbors_matrix.astype(jnp.int32)
    offs = (jnp.arange(B, dtype=jnp.int32) * N)[:, None, None]
    flat = (nbr + offs).reshape(_NODES, K)
    idx_arr = flat.reshape(_NW, _NCHUNK, _CHN, K).transpose(0, 1, 3, 2)

    c_ins = [N_SPACE + N_ALL] + [N_SPACE + d for d in LAYER_DIMS[:-1]]
    c_pads = [_pad8(c) for c in c_ins]
    w_outs = [N_SPACE + d for d in LAYER_DIMS]
    w_pads = c_pads[1:] + [_pad8(w_outs[-1])]

    cat = jnp.concatenate([space_features, all_features], axis=-1)
    cat = jnp.pad(cat, ((0, 0), (0, 0), (0, c_pads[0] - c_ins[0])))

    for i in range(len(LAYER_DIMS)):
        Wcat = jnp.concatenate([params["Ws%d" % i], params["Wa%d" % i]],
                               axis=1)
        bcat = jnp.concatenate([params["bs%d" % i], params["ba%d" % i]])
        Wp = _pad_weight(Wcat, c_ins[i], c_pads[i], w_outs[i], w_pads[i])
        bp = jnp.pad(bcat, (0, w_pads[i] - w_outs[i]))[None]

        gather = _make_sc_gather(c_pads[i], _NODES, _NCHUNK, _CHN)
        pre = gather(cat.reshape(_NODES, c_pads[i]), idx_arr)
        cat = _layer_matmul(pre.reshape(B, N, K * c_pads[i]), Wp, bp)

    return _head(cat[:, :, N_SPACE:N_SPACE + LAYER_DIMS[-1]], num_entries,
                 params["W_fc1"], params["b_fc1"][None],
                 params["W_fc2"], params["b_fc2"][None],
                 params["W_fc3"], params["b_fc3"][None])
